# Initial kernel scaffold; baseline (speedup 1.0000x reference)
import functools

import jax
import jax.numpy as jnp
from jax import lax
from jax.experimental import pallas as pl
from jax.experimental.pallas import tpu as pltpu
from jax.experimental.pallas import tpu_sc as plsc

_NC = 2
_NS = 16
_LANES = 16


def _mm_first(x, W):
    N, K = x.shape
    M = W.shape[1]
    BN = 1000

    def body(x_ref, w_ref, o_ref):
        o_ref[...] = jnp.dot(x_ref[...], w_ref[...],
                             preferred_element_type=jnp.float32)

    return pl.pallas_call(
        body,
        grid=(N // BN,),
        in_specs=[
            pl.BlockSpec((BN, K), lambda i: (i, 0)),
            pl.BlockSpec((K, M), lambda i: (0, 0)),
        ],
        out_specs=pl.BlockSpec((BN, M), lambda i: (i, 0)),
        out_shape=jax.ShapeDtypeStruct((N, M), jnp.float32),
    )(x, W)


def _mm_fused(parts, b, W):
    _, N, K = parts.shape
    M = W.shape[1]
    BN = 1000

    def body(p_ref, b_ref, w_ref, o_ref):
        h = jnp.maximum(p_ref[0] + p_ref[1] + b_ref[...], 0.0)
        o_ref[...] = jnp.dot(h, w_ref[...],
                             preferred_element_type=jnp.float32)

    return pl.pallas_call(
        body,
        grid=(N // BN,),
        in_specs=[
            pl.BlockSpec((2, BN, K), lambda i: (0, i, 0)),
            pl.BlockSpec((K,), lambda i: (0,)),
            pl.BlockSpec((K, M), lambda i: (0, 0)),
        ],
        out_specs=pl.BlockSpec((BN, M), lambda i: (i, 0)),
        out_shape=jax.ShapeDtypeStruct((N, M), jnp.float32),
    )(parts, b, W)


def _final_act(parts, b):
    _, N, K = parts.shape
    BN = 1000

    def body(p_ref, b_ref, o_ref):
        o_ref[...] = jnp.maximum(p_ref[0] + p_ref[1] + b_ref[...], 0.0)

    return pl.pallas_call(
        body,
        grid=(N // BN,),
        in_specs=[
            pl.BlockSpec((2, BN, K), lambda i: (0, i, 0)),
            pl.BlockSpec((K,), lambda i: (0,)),
        ],
        out_specs=pl.BlockSpec((BN, K), lambda i: (i, 0)),
        out_shape=jax.ShapeDtypeStruct((N, K), jnp.float32),
    )(parts, b)


def _sc_aggregate(h, src, dst, w, zeros):
    N, D = h.shape
    E = src.shape[0]
    NW = _NC * _NS
    e_per_w = E // NW
    CH = 80
    n_chunks = e_per_w // CH
    rows_per_tile = N // _NS

    mesh = plsc.VectorSubcoreMesh(core_axis_name="c", subcore_axis_name="s")

    @functools.partial(
        pl.kernel,
        mesh=mesh,
        out_type=jax.ShapeDtypeStruct((_NC, N, D), jnp.float32),
        scratch_types=[
            pltpu.VMEM((CH,), jnp.int32),
            pltpu.VMEM((CH,), jnp.int32),
            pltpu.VMEM((CH,), jnp.float32),
            pltpu.VMEM((CH, D), jnp.float32),
            pltpu.VMEM_SHARED((N, D), jnp.float32),
            pltpu.SemaphoreType.DMA,
        ],
    )
    def k(h_hbm, src_hbm, dst_hbm, w_hbm, z_hbm, out_hbm,
          src_v, dst_v, w_v, rows_v, acc_sp, sem):
        c = lax.axis_index("c")
        s = lax.axis_index("s")
        wid = s * _NC + c

        pltpu.sync_copy(z_hbm.at[pl.ds(s * rows_per_tile, rows_per_tile)],
                        acc_sp.at[pl.ds(s * rows_per_tile, rows_per_tile)])
        plsc.subcore_barrier()

        base_w = wid * e_per_w

        def chunk_body(i, carry):
            base = base_w + i * CH
            pltpu.sync_copy(src_hbm.at[pl.ds(base, CH)], src_v)
            pltpu.sync_copy(dst_hbm.at[pl.ds(base, CH)], dst_v)
            pltpu.sync_copy(w_hbm.at[pl.ds(base, CH)], w_v)
            pltpu.async_copy(h_hbm.at[src_v], rows_v, sem).wait()

            def edge_body(e, inner):
                wb = plsc.load_gather(w_v, [jnp.full((_LANES,), e, jnp.int32)])
                for j in range(D // _LANES):
                    sl = pl.ds(j * _LANES, _LANES)
                    rows_v[e, sl] = rows_v[e, sl] * wb
                return inner

            lax.fori_loop(0, CH, edge_body, 0)
            pltpu.sync_copy(rows_v, acc_sp.at[dst_v], add=True)
            return carry

        lax.fori_loop(0, n_chunks, chunk_body, 0)
        plsc.subcore_barrier()

        pltpu.sync_copy(acc_sp.at[pl.ds(s * rows_per_tile, rows_per_tile)],
                        out_hbm.at[c, pl.ds(s * rows_per_tile, rows_per_tile)])

    return k(h, src, dst, w, zeros)


def kernel(x, adj_index, adj_weight, W1, b1, W2, b2, W3, b3):
    src = adj_index[0].astype(jnp.int32)
    dst = adj_index[1].astype(jnp.int32)
    w = adj_weight.astype(jnp.float32)
    N, _ = x.shape
    D = W1.shape[1]
    zeros = jnp.zeros((N, D), jnp.float32)

    y = _mm_first(x, W1)
    p = _sc_aggregate(y, src, dst, w, zeros)
    y = _mm_fused(p, b1, W2)
    p = _sc_aggregate(y, src, dst, w, zeros)
    y = _mm_fused(p, b2, W3)
    p = _sc_aggregate(y, src, dst, w, zeros)
    return _final_act(p, b3)


# speedup vs baseline: 3.2612x; 3.2612x over previous
import functools

import jax
import jax.numpy as jnp
from jax import lax
from jax.experimental import pallas as pl
from jax.experimental.pallas import tpu as pltpu
from jax.experimental.pallas import tpu_sc as plsc

_NC = 2
_NS = 16
_LANES = 16


def _mm_first(x, W):
    N, K = x.shape
    M = W.shape[1]
    BN = 1000

    def body(x_ref, w_ref, o_ref):
        o_ref[...] = jnp.dot(x_ref[...], w_ref[...],
                             preferred_element_type=jnp.float32)

    return pl.pallas_call(
        body,
        grid=(N // BN,),
        in_specs=[
            pl.BlockSpec((BN, K), lambda i: (i, 0)),
            pl.BlockSpec((K, M), lambda i: (0, 0)),
        ],
        out_specs=pl.BlockSpec((BN, M), lambda i: (i, 0)),
        out_shape=jax.ShapeDtypeStruct((N, M), jnp.float32),
    )(x, W)


def _mm_fused(parts, b, W):
    _, N, K = parts.shape
    M = W.shape[1]
    BN = 1000

    def body(p_ref, b_ref, w_ref, o_ref):
        h = jnp.maximum(p_ref[0] + p_ref[1] + b_ref[...], 0.0)
        o_ref[...] = jnp.dot(h, w_ref[...],
                             preferred_element_type=jnp.float32)

    return pl.pallas_call(
        body,
        grid=(N // BN,),
        in_specs=[
            pl.BlockSpec((2, BN, K), lambda i: (0, i, 0)),
            pl.BlockSpec((K,), lambda i: (0,)),
            pl.BlockSpec((K, M), lambda i: (0, 0)),
        ],
        out_specs=pl.BlockSpec((BN, M), lambda i: (i, 0)),
        out_shape=jax.ShapeDtypeStruct((N, M), jnp.float32),
    )(parts, b, W)


def _final_act(parts, b):
    _, N, K = parts.shape
    BN = 1000

    def body(p_ref, b_ref, o_ref):
        o_ref[...] = jnp.maximum(p_ref[0] + p_ref[1] + b_ref[...], 0.0)

    return pl.pallas_call(
        body,
        grid=(N // BN,),
        in_specs=[
            pl.BlockSpec((2, BN, K), lambda i: (0, i, 0)),
            pl.BlockSpec((K,), lambda i: (0,)),
        ],
        out_specs=pl.BlockSpec((BN, K), lambda i: (i, 0)),
        out_shape=jax.ShapeDtypeStruct((N, K), jnp.float32),
    )(parts, b)


def _sc_aggregate(h, src, dst, w, zeros):
    N, D = h.shape
    E = src.shape[0]
    NW = _NC * _NS
    e_per_w = E // NW
    CH = 128
    n_chunks = e_per_w // CH
    R = (N // _NS) & ~7
    TAIL = N - _NS * R

    mesh = plsc.VectorSubcoreMesh(core_axis_name="c", subcore_axis_name="s",
                                  num_cores=_NC, num_subcores=_NS)

    @functools.partial(
        pl.kernel,
        mesh=mesh,
        out_type=jax.ShapeDtypeStruct((_NC, N, D), jnp.float32),
        compiler_params=pltpu.CompilerParams(needs_layout_passes=False),
        scratch_types=[
            pltpu.VMEM((CH,), jnp.int32),
            pltpu.VMEM((CH,), jnp.int32),
            pltpu.VMEM((CH,), jnp.float32),
            pltpu.VMEM((CH, D), jnp.float32),
            pltpu.VMEM_SHARED((N, D), jnp.float32),
            pltpu.SemaphoreType.DMA,
        ],
    )
    def k(h_hbm, src_hbm, dst_hbm, w_hbm, z_hbm, out_hbm,
          src_v, dst_v, w_v, rows_v, acc_sp, sem):
        c = lax.axis_index("c")
        s = lax.axis_index("s")
        wid = s * _NC + c

        pltpu.sync_copy(z_hbm.at[pl.ds(s * R, R)],
                        acc_sp.at[pl.ds(s * R, R)])

        @pl.when(s == 0)
        def _():
            pltpu.sync_copy(z_hbm.at[pl.ds(_NS * R, TAIL)],
                            acc_sp.at[pl.ds(_NS * R, TAIL)])

        plsc.subcore_barrier()

        base_w = wid * e_per_w

        def chunk_body(i, carry):
            base = base_w + i * CH
            pltpu.sync_copy(src_hbm.at[pl.ds(base, CH)], src_v)
            pltpu.sync_copy(dst_hbm.at[pl.ds(base, CH)], dst_v)
            pltpu.sync_copy(w_hbm.at[pl.ds(base, CH)], w_v)
            pltpu.async_copy(h_hbm.at[src_v], rows_v, sem).wait()

            def edge_body(e, inner):
                wb = plsc.load_gather(w_v, [jnp.full((_LANES,), e, jnp.int32)])
                for j in range(D // _LANES):
                    sl = pl.ds(j * _LANES, _LANES)
                    rows_v[e, sl] = rows_v[e, sl] * wb
                return inner

            lax.fori_loop(0, CH, edge_body, 0)
            pltpu.sync_copy(rows_v, acc_sp.at[dst_v], add=True)
            return carry

        lax.fori_loop(0, n_chunks, chunk_body, 0)
        plsc.subcore_barrier()

        pltpu.sync_copy(acc_sp.at[pl.ds(s * R, R)],
                        out_hbm.at[c, pl.ds(s * R, R)])

        @pl.when(s == 0)
        def _():
            pltpu.sync_copy(acc_sp.at[pl.ds(_NS * R, TAIL)],
                            out_hbm.at[c, pl.ds(_NS * R, TAIL)])

    return k(h, src, dst, w, zeros)


def kernel(x, adj_index, adj_weight, W1, b1, W2, b2, W3, b3):
    src = adj_index[0].astype(jnp.int32)
    dst = adj_index[1].astype(jnp.int32)
    w = adj_weight.astype(jnp.float32)
    N, _ = x.shape
    D = W1.shape[1]
    zeros = jnp.zeros((N, D), jnp.float32)

    E = src.shape[0]
    grain = _NC * _NS * 128
    E_pad = ((E + grain - 1) // grain) * grain
    if E_pad != E:
        pad = E_pad - E
        src = jnp.pad(src, (0, pad))
        dst = jnp.pad(dst, (0, pad))
        w = jnp.pad(w, (0, pad))

    y = _mm_first(x, W1)
    p = _sc_aggregate(y, src, dst, w, zeros)
    y = _mm_fused(p, b1, W2)
    p = _sc_aggregate(y, src, dst, w, zeros)
    y = _mm_fused(p, b2, W3)
    p = _sc_aggregate(y, src, dst, w, zeros)
    return _final_act(p, b3)
